# baseline (speedup 1.0000x reference)
import functools

import jax
import jax.numpy as jnp
from jax import lax
from jax.experimental import pallas as pl
from jax.experimental.pallas import tpu as pltpu
from jax.experimental.pallas import tpu_sc as plsc

_B, _L = 16384, 200
_N = _B * _L
_V, _D = 1000, 100
_VPAD = 1024

_NC, _NS = 2, 16
_NW = _NC * _NS
_CW = _B // _NW
_RB = 8
_NBLK = _L // _RB
_NBUF = 6

_sc_mesh = plsc.VectorSubcoreMesh(core_axis_name="c", subcore_axis_name="s")


@functools.partial(
    pl.kernel,
    mesh=_sc_mesh,
    out_type=jax.ShapeDtypeStruct((_L, _B // 1024, 8, 128), jnp.float32),
    scratch_types=[
        pltpu.VMEM((_VPAD,), jnp.float32),
        pltpu.VMEM((_D, 128), jnp.float32),
        pltpu.VMEM((_D + 12,), jnp.float32),
        pltpu.VMEM_SHARED((_VPAD,), jnp.float32),
        pltpu.VMEM((_NBUF, _RB, _CW), jnp.int32),
        pltpu.VMEM((_NBUF, _RB, 4, 128), jnp.float32),
        pltpu.SemaphoreType.DMA((_NBUF,)),
        pltpu.SemaphoreType.DMA((_NBUF,)),
    ],
    compiler_params=pltpu.CompilerParams(
        needs_layout_passes=False,
        use_tc_tiling_on_sc=True,
    ),
)
def _sc_kernel(wt_hbm, dwb_hbm, idx_hbm, out_hbm,
               table_v, wblk_v, dw_v, table_sh, idx_v, out_v,
               sem_in, sem_out):
    sid = lax.axis_index("s")
    wid = sid * _NC + lax.axis_index("c")
    c0 = wid * _CW
    k0 = wid // 2
    r0 = (wid % 2) * 4

    def in_copy(m, b):
        return pltpu.make_async_copy(
            idx_hbm.at[pl.ds(m * _RB, _RB), pl.ds(c0, _CW)],
            idx_v.at[b], sem_in.at[b])

    def out_copy(m, b):
        return pltpu.make_async_copy(
            out_v.at[b],
            out_hbm.at[pl.ds(m * _RB, _RB), k0, pl.ds(r0, 4), :],
            sem_out.at[b])

    for p in range(_NBUF):
        in_copy(p, p).start()

    pltpu.sync_copy(dwb_hbm, dw_v)

    @pl.when(sid < 8)
    def _build():
        v0 = sid * 128

        pltpu.sync_copy(wt_hbm.at[:, pl.ds(v0, 128)], wblk_v)

        bias = plsc.load_gather(dw_v, [jnp.full((16,), _D, jnp.int32)])
        acc0 = tuple(bias for _ in range(8))

        def fma(d, acc):
            w = plsc.load_gather(dw_v, [jnp.full((16,), d, jnp.int32)])
            return tuple(
                acc[j] + wblk_v[d, pl.ds(j * 16, 16)] * w for j in range(8))

        acc = lax.fori_loop(0, _D, fma, acc0, unroll=2)
        for j in range(8):
            table_v[pl.ds(j * 16, 16)] = acc[j]
        pltpu.sync_copy(table_v.at[pl.ds(0, 128)], table_sh.at[pl.ds(v0, 128)])

    plsc.subcore_barrier()
    pltpu.sync_copy(table_sh, table_v)

    def block(m, carry):
        b = lax.rem(m, _NBUF)
        in_copy(m, b).wait()

        @pl.when(m >= _NBUF)
        def _():
            out_copy(m - _NBUF, b).wait()

        @plsc.parallel_loop(0, _RB, unroll=2)
        def _gather_row(l):
            for j in range(_CW // 16):
                c = j * 16
                out_v[b, l, c // 128, pl.ds(c % 128, 16)] = plsc.load_gather(
                    table_v, [idx_v[b, l, pl.ds(c, 16)]])

        out_copy(m, b).start()

        @pl.when(m + _NBUF < _NBLK)
        def _():
            in_copy(m + _NBUF, b).start()

        return carry

    lax.fori_loop(0, _NBLK, block, 0)
    for p in range(_NBUF):
        m = _NBLK - _NBUF + p
        out_copy(m, m % _NBUF).wait()


def kernel(x, embd_weight, dense_W, dense_b):
    w_pad = jnp.pad(embd_weight, ((0, _VPAD - _V), (0, 0)))
    wt = jnp.swapaxes(w_pad, 0, 1)
    dwb = jnp.concatenate(
        [dense_W.reshape(_D), dense_b, jnp.zeros((11,), jnp.float32)])
    idx_t = jnp.swapaxes(x.astype(jnp.int32), 0, 1)
    out4 = _sc_kernel(wt, dwb, idx_t)
    out = jnp.transpose(out4, (1, 2, 3, 0))
    return out.reshape(_B, _L, 1)


# speedup vs baseline: 1.0135x; 1.0135x over previous
import functools

import jax
import jax.numpy as jnp
from jax import lax
from jax.experimental import pallas as pl
from jax.experimental.pallas import tpu as pltpu
from jax.experimental.pallas import tpu_sc as plsc

_B, _L = 16384, 200
_N = _B * _L
_V, _D = 1000, 100
_VPAD = 1024

_NC, _NS = 2, 16
_NW = _NC * _NS
_CW = _B // _NW
_RB = 8
_NBLK = _L // _RB
_NBUF = 4

_sc_mesh = plsc.VectorSubcoreMesh(core_axis_name="c", subcore_axis_name="s")


@functools.partial(
    pl.kernel,
    mesh=_sc_mesh,
    out_type=jax.ShapeDtypeStruct((_L, _B // 1024, 8, 128), jnp.float32),
    scratch_types=[
        pltpu.VMEM((_VPAD,), jnp.float32),
        pltpu.VMEM((_D, 128), jnp.float32),
        pltpu.VMEM((_D + 12,), jnp.float32),
        pltpu.VMEM_SHARED((_VPAD,), jnp.float32),
        pltpu.VMEM((_NBUF, _RB, _CW), jnp.int32),
        pltpu.VMEM((_NBUF, _RB, 4, 128), jnp.float32),
        pltpu.SemaphoreType.DMA((_NBUF,)),
        pltpu.SemaphoreType.DMA((_NBUF,)),
    ],
    compiler_params=pltpu.CompilerParams(
        needs_layout_passes=False,
        use_tc_tiling_on_sc=True,
        disable_bounds_checks=True,
        skip_device_barrier=True,
    ),
)
def _sc_kernel(wt_hbm, dwb_hbm, idx_hbm, out_hbm,
               table_v, wblk_v, dw_v, table_sh, idx_v, out_v,
               sem_in, sem_out):
    sid = lax.axis_index("s")
    wid = sid * _NC + lax.axis_index("c")
    c0 = wid * _CW
    k0 = wid // 2
    r0 = (wid % 2) * 4

    def in_copy(m, b):
        return pltpu.make_async_copy(
            idx_hbm.at[pl.ds(m * _RB, _RB), pl.ds(c0, _CW)],
            idx_v.at[b], sem_in.at[b])

    def out_copy(m, b):
        return pltpu.make_async_copy(
            out_v.at[b],
            out_hbm.at[pl.ds(m * _RB, _RB), k0, pl.ds(r0, 4), :],
            sem_out.at[b])

    for p in range(_NBUF):
        in_copy(p, p).start()

    pltpu.sync_copy(dwb_hbm, dw_v)

    @pl.when(sid < 8)
    def _build():
        v0 = sid * 128

        pltpu.sync_copy(wt_hbm.at[:, pl.ds(v0, 128)], wblk_v)

        bias = plsc.load_gather(dw_v, [jnp.full((16,), _D, jnp.int32)])
        acc0 = tuple(bias for _ in range(8))

        def fma(d, acc):
            w = plsc.load_gather(dw_v, [jnp.full((16,), d, jnp.int32)])
            return tuple(
                acc[j] + wblk_v[d, pl.ds(j * 16, 16)] * w for j in range(8))

        acc = lax.fori_loop(0, _D, fma, acc0, unroll=2)
        for j in range(8):
            table_v[pl.ds(j * 16, 16)] = acc[j]
        pltpu.sync_copy(table_v.at[pl.ds(0, 128)], table_sh.at[pl.ds(v0, 128)])

    plsc.subcore_barrier()
    pltpu.sync_copy(table_sh, table_v)

    def block(m, carry):
        b = lax.rem(m, _NBUF)
        in_copy(m, b).wait()

        @pl.when(m >= _NBUF)
        def _():
            out_copy(m - _NBUF, b).wait()

        @plsc.parallel_loop(0, _RB, unroll=2)
        def _gather_row(l):
            for j in range(_CW // 16):
                c = j * 16
                out_v[b, l, c // 128, pl.ds(c % 128, 16)] = plsc.load_gather(
                    table_v, [idx_v[b, l, pl.ds(c, 16)]])

        out_copy(m, b).start()

        @pl.when(m + _NBUF < _NBLK)
        def _():
            in_copy(m + _NBUF, b).start()

        return carry

    lax.fori_loop(0, _NBLK, block, 0)
    for p in range(_NBUF):
        m = _NBLK - _NBUF + p
        out_copy(m, m % _NBUF).wait()


def kernel(x, embd_weight, dense_W, dense_b):
    w_pad = jnp.pad(embd_weight, ((0, _VPAD - _V), (0, 0)))
    wt = jnp.swapaxes(w_pad, 0, 1)
    dwb = jnp.concatenate(
        [dense_W.reshape(_D), dense_b, jnp.zeros((11,), jnp.float32)])
    idx_t = jnp.swapaxes(x.astype(jnp.int32), 0, 1)
    out4 = _sc_kernel(wt, dwb, idx_t)
    out = jnp.transpose(out4, (1, 2, 3, 0))
    return out.reshape(_B, _L, 1)
